# baseline (speedup 1.0000x reference)
import functools

import jax
import jax.numpy as jnp
from jax import lax
from jax.experimental import pallas as pl
from jax.experimental.pallas import tpu as pltpu
from jax.experimental.pallas import tpu_sc as plsc

VOCAB = 30522
D = 768
BATCH = 128
SEQ = 512
B = BATCH * SEQ

NC = 2
NS = 16
NW = NC * NS
LANES = 16

ROWS_PER_W = B // NW
CHUNK = 64
N_CHUNKS = ROWS_PER_W // CHUNK
VREGS_PER_ROW = D // LANES


def _sc_body(seq_hbm, table_hbm, pos_hbm, out_hbm, idx_v, rows_v, pos_v, sem):
    wid = lax.axis_index("s") * NC + lax.axis_index("c")
    base = wid * ROWS_PER_W

    def chunk_step(c, carry):
        off = base + c * CHUNK
        s_off = lax.rem(off, SEQ)
        pltpu.sync_copy(seq_hbm.at[pl.ds(off, CHUNK)], idx_v)
        pltpu.async_copy(table_hbm.at[idx_v], rows_v, sem).wait()
        pltpu.sync_copy(pos_hbm.at[pl.ds(s_off, CHUNK)], pos_v)

        def row_step(i, carry2):
            for j in range(VREGS_PER_ROW):
                g = rows_v[i, pl.ds(j * LANES, LANES)]
                p = pos_v[i, pl.ds(j * LANES, LANES)]
                rows_v[i, pl.ds(j * LANES, LANES)] = (g + p) * 2.0
            return carry2

        lax.fori_loop(0, CHUNK, row_step, 0, unroll=False)
        pltpu.sync_copy(rows_v, out_hbm.at[pl.ds(off, CHUNK)])
        return carry

    lax.fori_loop(0, N_CHUNKS, chunk_step, 0, unroll=False)


@jax.jit
def _embed(seq_flat, content_table, pos_pe):
    mesh = plsc.VectorSubcoreMesh(core_axis_name="c", subcore_axis_name="s")
    k = functools.partial(
        pl.kernel,
        mesh=mesh,
        out_type=jax.ShapeDtypeStruct((B, D), jnp.float32),
        scratch_types=[
            pltpu.VMEM((CHUNK,), jnp.int32),
            pltpu.VMEM((CHUNK, D), jnp.float32),
            pltpu.VMEM((CHUNK, D), jnp.float32),
            pltpu.SemaphoreType.DMA,
        ],
    )(_sc_body)
    return k(seq_flat, content_table, pos_pe)


def kernel(sequence, content_table, pos_pe):
    seq_flat = sequence.reshape(B)
    out = _embed(seq_flat, content_table, pos_pe)
    return out.reshape(BATCH, SEQ, D)
